# baseline (speedup 1.0000x reference)
import functools

import jax
import jax.numpy as jnp
from jax import lax
from jax.experimental import pallas as pl
from jax.experimental.pallas import tpu as pltpu
from jax.experimental.pallas import tpu_sc as plsc

_XMIN = 0.0
_YMIN = 0.0
_RESOLUTION = 0.000244140625
_INV_RES = 1.0 / _RESOLUTION

_GX = 4096
_GY = 4096
_B = 16384
_NC = 2
_NS = 16
_NW = _NC * _NS
_BPW = _B // _NW
_L = 16
_CHUNK = 128
_PAD = _CHUNK
_OUT = _B + _PAD

_mesh = plsc.VectorSubcoreMesh(core_axis_name="c", subcore_axis_name="s")
_params = pltpu.CompilerParams(needs_layout_passes=False)


def _index_body(xc_hbm, yc_hbm, rc_hbm, xv, yv, rcv):
    wid = lax.axis_index("s") * _NC + lax.axis_index("c")
    base = wid * _BPW

    pltpu.sync_copy(xc_hbm.at[pl.ds(base, _BPW)], xv)
    pltpu.sync_copy(yc_hbm.at[pl.ds(base, _BPW)], yv)

    def _idx(i, _):
        sl = pl.ds(i * _L, _L)
        ix = jnp.minimum(
            jnp.maximum((xv[sl] * _INV_RES).astype(jnp.int32), 0), _GX - 1)
        iy = jnp.minimum(
            jnp.maximum((yv[sl] * _INV_RES).astype(jnp.int32), 0), _GY - 1)
        rcv[sl] = (ix << 12) | iy
        return _
    lax.fori_loop(0, _BPW // _L, _idx, None)

    pltpu.sync_copy(rcv, rc_hbm.at[pl.ds(base, _BPW)])


def _gather_body(
    rc_hbm, grid_hbm, out_hbm,
    rcall, rowbuf, metabuf, resv, valbuf, sem, sem2,
):
    wid = lax.axis_index("s") * _NC + lax.axis_index("c")

    pltpu.sync_copy(rc_hbm, rcall)

    zeros = jnp.zeros((_L,), jnp.int32)
    lanes = lax.iota(jnp.int32, _L)

    def _scan(g, bsplat):
        rc16 = rcall[pl.ds(g * _L, _L)]
        m = ((rc16 >> 7) & 31) == wid
        pos16 = bsplat + plsc.cumsum(jnp.where(m, 1, 0)) - 1
        plsc.store_scatter(rowbuf, [pos16], rc16 >> 12, mask=m)
        meta16 = ((g * _L + lanes) << 7) | (rc16 & 127)
        plsc.store_scatter(metabuf, [pos16], meta16, mask=m)
        return bsplat + plsc.all_reduce_population_count(m)
    bsplat = lax.fori_loop(0, _B // _L, _scan, zeros)
    n = jnp.max(bsplat)

    for g in range(_CHUNK // _L):
        p16 = bsplat + (g * _L) + lanes
        plsc.store_scatter(rowbuf, [p16], (p16 * 8) & (_GX - 1))
        plsc.store_scatter(metabuf, [p16], zeros + ((_B + (wid * 4)) << 7))

    cw = pl.multiple_of(wid * _CHUNK, _CHUNK)
    nch = (n + _CHUNK - 1) >> 7

    def _chunk(d, _):
        off = pl.multiple_of(d * _CHUNK, _CHUNK)
        cp = pltpu.make_async_copy(
            grid_hbm.at[rowbuf.at[pl.ds(off, _CHUNK)], pl.ds(cw, _CHUNK)],
            resv,
            sem,
        )
        cp.start()
        cp.wait()

        scat = []
        for g in range(_CHUNK // _L):
            sl = pl.ds(g * _L, _L)
            meta16 = metabuf[pl.ds(off + g * _L, _L)]
            vals = plsc.load_gather(resv, [lanes + g * _L, meta16 & 127])
            valbuf[sl] = vals
            scat.append(pltpu.make_async_copy(
                valbuf.at[sl], out_hbm.at[meta16 >> 7], sem2))
        for cp2 in scat:
            cp2.start()
        for cp2 in scat:
            cp2.wait()
        return _
    lax.fori_loop(0, nch, _chunk, None)


@jax.jit
def kernel(x, grid):
    xc = x[:, 0]
    yc = x[:, 1]

    index_run = pl.kernel(
        _index_body,
        out_type=jax.ShapeDtypeStruct((_B,), jnp.int32),
        mesh=_mesh,
        compiler_params=_params,
        scratch_types=[
            pltpu.VMEM((_BPW,), jnp.float32),
            pltpu.VMEM((_BPW,), jnp.float32),
            pltpu.VMEM((_BPW,), jnp.int32),
        ],
    )
    rc = index_run(xc, yc)

    gather_run = pl.kernel(
        _gather_body,
        out_type=jax.ShapeDtypeStruct((_OUT,), jnp.float32),
        mesh=_mesh,
        compiler_params=_params,
        scratch_types=[
            pltpu.VMEM((_B,), jnp.int32),
            pltpu.VMEM((_B + _PAD,), jnp.int32),
            pltpu.VMEM((_B + _PAD,), jnp.int32),
            pltpu.VMEM((_CHUNK, _CHUNK), jnp.float32),
            pltpu.VMEM((_CHUNK,), jnp.float32),
            pltpu.SemaphoreType.DMA,
            pltpu.SemaphoreType.DMA,
        ],
    )
    out = gather_run(rc, grid)
    return out[:_B]


# speedup vs baseline: 4.7508x; 4.7508x over previous
import functools

import jax
import jax.numpy as jnp
from jax import lax
from jax.experimental import pallas as pl
from jax.experimental.pallas import tpu as pltpu
from jax.experimental.pallas import tpu_sc as plsc

_XMIN = 0.0
_YMIN = 0.0
_RESOLUTION = 0.000244140625
_INV_RES = 1.0 / _RESOLUTION

_GX = 4096
_GY = 4096
_B = 16384
_NC = 2
_NS = 16
_NW = _NC * _NS
_BPW = _B // _NW
_L = 16
_CHUNK = 128
_NCH = _BPW // _CHUNK


def _grid_gather_body(x_hbm, gflat_hbm, out_hbm, xyv, idxv, resv, outv, sem):
    wid = lax.axis_index("s") * _NC + lax.axis_index("c")
    base = wid * _BPW

    pltpu.sync_copy(x_hbm.at[pl.ds(base, _BPW), :], xyv)

    lanes = lax.iota(jnp.int32, _L)
    zeros = jnp.zeros((_L,), jnp.int32)
    ones = zeros + 1

    def _idx(i, _):
        rows = lanes + i * _L
        xs = plsc.load_gather(xyv, [rows, zeros])
        ys = plsc.load_gather(xyv, [rows, ones])
        ix = jnp.minimum(
            jnp.maximum((xs * _INV_RES).astype(jnp.int32), 0), _GX - 1)
        iy = jnp.minimum(
            jnp.maximum((ys * _INV_RES).astype(jnp.int32), 0), _GY - 1)
        k, o = divmod(i, _CHUNK // _L)
        idxv[k, pl.ds(o * _L, _L)] = (ix << 12) | iy
        return _
    for i in range(_BPW // _L):
        _idx(i, None)

    copies = [
        pltpu.make_async_copy(gflat_hbm.at[idxv.at[j]], resv.at[j], sem)
        for j in range(_NCH)
    ]
    for cp in copies:
        cp.start()
    for cp in copies:
        cp.wait()

    for j in range(_NCH):
        pltpu.sync_copy(resv.at[j], out_hbm.at[pl.ds(base + j * _CHUNK, _CHUNK)])


@jax.jit
def kernel(x, grid):
    gflat = grid.reshape(-1)

    mesh = plsc.VectorSubcoreMesh(core_axis_name="c", subcore_axis_name="s")
    run = pl.kernel(
        _grid_gather_body,
        out_type=jax.ShapeDtypeStruct((_B,), jnp.float32),
        mesh=mesh,
        compiler_params=pltpu.CompilerParams(needs_layout_passes=False),
        scratch_types=[
            pltpu.VMEM((_BPW, 2), jnp.float32),
            pltpu.VMEM((_NCH, _CHUNK), jnp.int32),
            pltpu.VMEM((_NCH, _CHUNK), jnp.float32),
            pltpu.VMEM((_BPW,), jnp.float32),
            pltpu.SemaphoreType.DMA,
        ],
    )
    return run(x, gflat)


# speedup vs baseline: 5.0826x; 1.0698x over previous
import functools

import jax
import jax.numpy as jnp
from jax import lax
from jax.experimental import pallas as pl
from jax.experimental.pallas import tpu as pltpu
from jax.experimental.pallas import tpu_sc as plsc

_XMIN = 0.0
_YMIN = 0.0
_RESOLUTION = 0.000244140625
_INV_RES = 1.0 / _RESOLUTION

_GX = 4096
_GY = 4096
_B = 16384
_NC = 2
_NS = 16
_NW = _NC * _NS
_BPW = _B // _NW
_L = 16
_CHUNK = 128
_NCH = _BPW // _CHUNK


def _grid_gather_body(xc_hbm, yc_hbm, gflat_hbm, out_hbm, xv, yv, idxv, resv, sem):
    wid = lax.axis_index("s") * _NC + lax.axis_index("c")
    base = wid * _BPW

    pltpu.sync_copy(xc_hbm.at[pl.ds(base, _BPW)], xv)
    pltpu.sync_copy(yc_hbm.at[pl.ds(base, _BPW)], yv)

    for i in range(_BPW // _L):
        xs = xv[pl.ds(i * _L, _L)]
        ys = yv[pl.ds(i * _L, _L)]
        ix = jnp.minimum(
            jnp.maximum((xs * _INV_RES).astype(jnp.int32), 0), _GX - 1)
        iy = jnp.minimum(
            jnp.maximum((ys * _INV_RES).astype(jnp.int32), 0), _GY - 1)
        flat = ix * _GY + iy
        idxv[i // (_CHUNK // _L), pl.ds((i % (_CHUNK // _L)) * _L, _L)] = flat

    copies = [
        pltpu.make_async_copy(gflat_hbm.at[idxv.at[j]], resv.at[j], sem)
        for j in range(_NCH)
    ]
    for cp in copies:
        cp.start()
    for cp in copies:
        cp.wait()

    for j in range(_NCH):
        pltpu.sync_copy(resv.at[j], out_hbm.at[pl.ds(base + j * _CHUNK, _CHUNK)])


@jax.jit
def kernel(x, grid):
    xc = x[:, 0]
    yc = x[:, 1]
    gflat = grid.reshape(-1)

    mesh = plsc.VectorSubcoreMesh(core_axis_name="c", subcore_axis_name="s")
    run = pl.kernel(
        _grid_gather_body,
        out_type=jax.ShapeDtypeStruct((_B,), jnp.float32),
        mesh=mesh,
        scratch_types=[
            pltpu.VMEM((_BPW,), jnp.float32),
            pltpu.VMEM((_BPW,), jnp.float32),
            pltpu.VMEM((_NCH, _CHUNK), jnp.int32),
            pltpu.VMEM((_NCH, _CHUNK), jnp.float32),
            pltpu.SemaphoreType.DMA,
        ],
    )
    return run(xc, yc, gflat)
